# Initial kernel scaffold; baseline (speedup 1.0000x reference)
import jax
import jax.numpy as jnp
from jax.experimental import pallas as pl


def kernel(x, edge_index, W_in, W_mid, W_out):
    raise NotImplementedError("write your pallas kernel here")


# speedup vs baseline: 27.2899x; 27.2899x over previous
import functools

import jax
import jax.numpy as jnp
from jax import lax
from jax.experimental import pallas as pl
from jax.experimental.pallas import tpu as pltpu
from jax.experimental.pallas import tpu_sc as plsc

N = 10000
E = 320000
H = 16
NC = 2
NS = 16
NW = NC * NS
EPT = E // NW
ROW_STRIDE = 624
ROWS_PT = 640
K = 2000
NCH = EPT // K

_mesh = plsc.VectorSubcoreMesh(
    core_axis_name="c", subcore_axis_name="s", num_cores=NC, num_subcores=NS
)


@functools.partial(
    pl.kernel,
    out_type=[
        jax.ShapeDtypeStruct((N, H), jnp.float32),
        jax.ShapeDtypeStruct((N, H), jnp.float32),
    ],
    mesh=_mesh,
    scratch_types=[
        pltpu.VMEM((K,), jnp.int32),
        pltpu.VMEM((K,), jnp.int32),
        pltpu.VMEM((K, H), jnp.float32),
        pltpu.VMEM((ROWS_PT, H), jnp.float32),
        pltpu.VMEM_SHARED((N, H), jnp.float32),
        pltpu.SemaphoreType.DMA,
    ],
    compiler_params=pltpu.CompilerParams(use_tc_tiling_on_sc=False),
)
def _sc_aggregate(g_hbm, src_hbm, dst_hbm, zrows_hbm, p0_hbm, p1_hbm,
                  sidx, didx, rows, wb, acc, sem):
    c = lax.axis_index("c")
    t = lax.axis_index("s")
    r0 = t * ROW_STRIDE
    pltpu.sync_copy(zrows_hbm, wb)
    pltpu.sync_copy(wb, acc.at[pl.ds(r0, ROWS_PT)])
    plsc.subcore_barrier()
    base = (c * NS + t) * EPT

    @pl.loop(0, NCH)
    def _chunk(i):
        off = base + i * K
        pltpu.sync_copy(src_hbm.at[pl.ds(off, K)], sidx)
        pltpu.sync_copy(dst_hbm.at[pl.ds(off, K)], didx)
        pltpu.async_copy(g_hbm.at[sidx], rows, sem).wait()
        pltpu.sync_copy(rows, acc.at[didx], add=True)

    plsc.subcore_barrier()
    pltpu.sync_copy(acc.at[pl.ds(r0, ROWS_PT)], wb)

    @pl.when(c == 0)
    def _():
        pltpu.sync_copy(wb, p0_hbm.at[pl.ds(r0, ROWS_PT)])

    @pl.when(c == 1)
    def _():
        pltpu.sync_copy(wb, p1_hbm.at[pl.ds(r0, ROWS_PT)])


def _prep_body(x_ref, w_ref, d0_ref, d1_ref, r_ref, g_ref):
    deg = jnp.maximum(d0_ref[...] + d1_ref[...], 1.0)
    r = lax.rsqrt(deg)
    r_ref[...] = r
    g_ref[...] = r * jnp.dot(x_ref[...], w_ref[...],
                             preferred_element_type=jnp.float32)


def _inter_body(p0_ref, p1_ref, r_ref, w_ref, g_ref):
    r = r_ref[...]
    h = jnp.maximum(r * (p0_ref[...] + p1_ref[...]), 0.0)
    g_ref[...] = r * jnp.dot(h, w_ref[...], preferred_element_type=jnp.float32)


def _final_body(p0_ref, p1_ref, r_ref, o_ref):
    z = r_ref[...] * (p0_ref[...] + p1_ref[...])
    col = lax.broadcasted_iota(jnp.int32, z.shape, 1)
    zm = jnp.where(col < 10, z, -jnp.inf)
    m = jnp.max(zm, axis=1, keepdims=True)
    s = jnp.sum(jnp.exp(zm - m), axis=1, keepdims=True)
    out = z - m - jnp.log(s)
    o_ref[...] = out[:, :10]


def kernel(x, edge_index, W_in, W_mid, W_out):
    f32 = jnp.float32
    src = edge_index[0].astype(jnp.int32)
    dst = edge_index[1].astype(jnp.int32)
    zrows = jnp.zeros((ROWS_PT, H), dtype=f32)
    ones_tab = jnp.ones((N, H), dtype=f32)
    w_out_pad = jnp.zeros((H, H), dtype=f32).at[:, :10].set(W_out)

    d0, d1 = _sc_aggregate(ones_tab, src, dst, zrows)

    prep = pl.pallas_call(
        _prep_body,
        out_shape=[jax.ShapeDtypeStruct((N, H), f32),
                   jax.ShapeDtypeStruct((N, H), f32)],
    )
    r_tab, g = prep(x, W_in, d0, d1)

    inter = pl.pallas_call(
        _inter_body,
        out_shape=jax.ShapeDtypeStruct((N, H), f32),
    )
    for i in range(W_mid.shape[0]):
        p0, p1 = _sc_aggregate(g, src, dst, zrows)
        g = inter(p0, p1, r_tab, W_mid[i])
    p0, p1 = _sc_aggregate(g, src, dst, zrows)
    g = inter(p0, p1, r_tab, w_out_pad)
    p0, p1 = _sc_aggregate(g, src, dst, zrows)

    final = pl.pallas_call(
        _final_body,
        out_shape=jax.ShapeDtypeStruct((N, 10), f32),
    )
    return final(p0, p1, r_tab)


# speedup vs baseline: 30.0254x; 1.1002x over previous
import functools

import jax
import jax.numpy as jnp
from jax import lax
from jax.experimental import pallas as pl
from jax.experimental.pallas import tpu as pltpu
from jax.experimental.pallas import tpu_sc as plsc

N = 10000
E = 320000
H = 16
NC = 2
NS = 16
NW = NC * NS
EPT = E // NW
ROW_STRIDE = 624
ROWS_PT = 640
K = 2000
NCH = EPT // K

_mesh = plsc.VectorSubcoreMesh(
    core_axis_name="c", subcore_axis_name="s", num_cores=NC, num_subcores=NS
)


@functools.partial(
    pl.kernel,
    out_type=[
        jax.ShapeDtypeStruct((N, H), jnp.float32),
        jax.ShapeDtypeStruct((N, H), jnp.float32),
    ],
    mesh=_mesh,
    scratch_types=[
        pltpu.VMEM((NCH, K), jnp.int32),
        pltpu.VMEM((NCH, K), jnp.int32),
        pltpu.VMEM((2, K, H), jnp.float32),
        pltpu.VMEM((ROWS_PT, H), jnp.float32),
        pltpu.VMEM_SHARED((N, H), jnp.float32),
        pltpu.SemaphoreType.DMA,
        pltpu.SemaphoreType.DMA,
    ],
    compiler_params=pltpu.CompilerParams(use_tc_tiling_on_sc=False),
)
def _sc_aggregate(g_hbm, src_hbm, dst_hbm, zrows_hbm, p0_hbm, p1_hbm,
                  sidx, didx, rows, wb, acc, sem0, sem1):
    c = lax.axis_index("c")
    t = lax.axis_index("s")
    r0 = t * ROW_STRIDE
    base = (c * NS + t) * EPT
    for i in range(NCH):
        pltpu.sync_copy(src_hbm.at[pl.ds(base + i * K, K)], sidx.at[i])
        pltpu.sync_copy(dst_hbm.at[pl.ds(base + i * K, K)], didx.at[i])
    pltpu.sync_copy(zrows_hbm, wb)
    pltpu.sync_copy(wb, acc.at[pl.ds(r0, ROWS_PT)])
    plsc.subcore_barrier()

    sems = (sem0, sem1)
    pltpu.async_copy(g_hbm.at[sidx.at[0]], rows.at[0], sems[0])
    for i in range(NCH):
        if i + 1 < NCH:
            pltpu.async_copy(g_hbm.at[sidx.at[i + 1]], rows.at[(i + 1) % 2],
                             sems[(i + 1) % 2])
        pltpu.make_async_copy(g_hbm.at[sidx.at[i]], rows.at[i % 2],
                              sems[i % 2]).wait()
        pltpu.sync_copy(rows.at[i % 2], acc.at[didx.at[i]], add=True)

    plsc.subcore_barrier()
    pltpu.sync_copy(acc.at[pl.ds(r0, ROWS_PT)], wb)

    @pl.when(c == 0)
    def _():
        pltpu.sync_copy(wb, p0_hbm.at[pl.ds(r0, ROWS_PT)])

    @pl.when(c == 1)
    def _():
        pltpu.sync_copy(wb, p1_hbm.at[pl.ds(r0, ROWS_PT)])


def _prep_body(x_ref, w_ref, d0_ref, d1_ref, r_ref, g_ref):
    deg = jnp.maximum(d0_ref[...] + d1_ref[...], 1.0)
    r = lax.rsqrt(deg)
    r_ref[...] = r
    g_ref[...] = r * jnp.dot(x_ref[...], w_ref[...],
                             preferred_element_type=jnp.float32)


def _inter_body(p0_ref, p1_ref, r_ref, w_ref, g_ref):
    r = r_ref[...]
    h = jnp.maximum(r * (p0_ref[...] + p1_ref[...]), 0.0)
    g_ref[...] = r * jnp.dot(h, w_ref[...], preferred_element_type=jnp.float32)


def _final_body(p0_ref, p1_ref, r_ref, o_ref):
    z = r_ref[...] * (p0_ref[...] + p1_ref[...])
    col = lax.broadcasted_iota(jnp.int32, z.shape, 1)
    zm = jnp.where(col < 10, z, -jnp.inf)
    m = jnp.max(zm, axis=1, keepdims=True)
    s = jnp.sum(jnp.exp(zm - m), axis=1, keepdims=True)
    out = z - m - jnp.log(s)
    o_ref[...] = out[:, :10]


def kernel(x, edge_index, W_in, W_mid, W_out):
    f32 = jnp.float32
    src = edge_index[0].astype(jnp.int32)
    dst = edge_index[1].astype(jnp.int32)
    zrows = jnp.zeros((ROWS_PT, H), dtype=f32)
    ones_tab = jnp.ones((N, H), dtype=f32)
    w_out_pad = jnp.zeros((H, H), dtype=f32).at[:, :10].set(W_out)

    d0, d1 = _sc_aggregate(ones_tab, src, dst, zrows)

    prep = pl.pallas_call(
        _prep_body,
        out_shape=[jax.ShapeDtypeStruct((N, H), f32),
                   jax.ShapeDtypeStruct((N, H), f32)],
    )
    r_tab, g = prep(x, W_in, d0, d1)

    inter = pl.pallas_call(
        _inter_body,
        out_shape=jax.ShapeDtypeStruct((N, H), f32),
    )
    for i in range(W_mid.shape[0]):
        p0, p1 = _sc_aggregate(g, src, dst, zrows)
        g = inter(p0, p1, r_tab, W_mid[i])
    p0, p1 = _sc_aggregate(g, src, dst, zrows)
    g = inter(p0, p1, r_tab, w_out_pad)
    p0, p1 = _sc_aggregate(g, src, dst, zrows)

    final = pl.pallas_call(
        _final_body,
        out_shape=jax.ShapeDtypeStruct((N, 10), f32),
    )
    return final(p0, p1, r_tab)
